# baseline (speedup 1.0000x reference)
import functools

import jax
import jax.numpy as jnp
from jax import lax
from jax.experimental import pallas as pl
from jax.experimental.pallas import tpu as pltpu
from jax.experimental.pallas import tpu_sc as plsc

B = 4096
L = 200
H = 64
NC = 2
NS = 16
NW = NC * NS
SW = B // NW
IC0 = 128
IC1 = L - IC0
NG = L // 16


def _make_kernel():
    mesh = plsc.VectorSubcoreMesh(core_axis_name="c", subcore_axis_name="s")

    @functools.partial(
        pl.kernel,
        mesh=mesh,
        out_type=jax.ShapeDtypeStruct((B, L, 2 * H), jnp.float32),
        scratch_types=[
            pltpu.VMEM((SW * L,), jnp.int32),
            pltpu.VMEM((2, L), jnp.int32),
            pltpu.VMEM((2, L, 2 * H), jnp.float32),
            pltpu.VMEM((2, L, 2 * H), jnp.float32),
            pltpu.SemaphoreType.DMA,
            pltpu.SemaphoreType.DMA,
            pltpu.SemaphoreType.DMA,
            pltpu.SemaphoreType.DMA,
        ],
    )
    def embed(idx_hbm, table_hbm, pre_hbm, out_hbm, idx_v, pidx_v, rows_v,
              out_v, sem_g0, sem_g1, sem_w0, sem_w1):
        wid = lax.axis_index("c") * NS + lax.axis_index("s")
        base = wid * SW
        sem_g = (sem_g0, sem_g1)
        sem_w = (sem_w0, sem_w1)

        pltpu.sync_copy(idx_hbm.at[pl.ds(base * L, SW * L)], idx_v)

        for b in range(2):
            pltpu.sync_copy(pre_hbm, out_v.at[b])

        def group_starts():
            return [g * 16 for g in range(NG)] + [L - 16]

        def build_pair_list(s, b):
            for r0 in group_starts():
                v = idx_v[pl.ds(s * L + r0, 16)]
                pidx_v[b, pl.ds(r0, 16)] = lax.shift_right_logical(v, 1)

        def gather_copies(s, b):
            return (
                pltpu.make_async_copy(
                    table_hbm.at[pidx_v.at[b].at[pl.ds(0, IC0)]],
                    rows_v.at[b].at[pl.ds(0, IC0)],
                    sem_g[b],
                ),
                pltpu.make_async_copy(
                    table_hbm.at[pidx_v.at[b].at[pl.ds(IC0, IC1)]],
                    rows_v.at[b].at[pl.ds(IC0, IC1)],
                    sem_g[b],
                ),
            )

        def wb_copy(s, b):
            return pltpu.make_async_copy(
                out_v.at[b],
                out_hbm.at[base + s],
                sem_w[b],
            )

        def issue_gathers(s, b):
            build_pair_list(s, b)
            for c in gather_copies(s, b):
                c.start()

        def interleave(s, b):
            def group(r0):
                hv = (idx_v[pl.ds(s * L + r0, 16)] & 1) * H
                for k in range(16):
                    row = r0 + k
                    off = hv[k]
                    for j in range(H // 16):
                        out_v[b, row, pl.ds(H + j * 16, 16)] = (
                            rows_v[b, row, pl.ds(off + j * 16, 16)]
                        )

            def il(g, carry):
                group(g * 16)
                return carry

            lax.fori_loop(0, NG, il, 0)
            group(L - 16)

        issue_gathers(0, 0)
        issue_gathers(1, 1)

        for b in range(2):
            for c in gather_copies(b, b):
                c.wait()
            interleave(b, b)
            wb_copy(b, b).start()
            issue_gathers(b + 2, b)

        def pair_body(g, carry):
            for b in range(2):
                s = 2 * g + b
                for c in gather_copies(s, b):
                    c.wait()
                wb_copy(s - 2, b).wait()
                interleave(s, b)
                wb_copy(s, b).start()

                @pl.when(s + 2 < SW)
                def _():
                    issue_gathers(s + 2, b)

            return carry

        lax.fori_loop(1, SW // 2, pair_body, 0)

        for b in range(2):
            wb_copy(SW - 2 + b, b).wait()

    return embed


_embed = _make_kernel()


def kernel(inputs, embedding, position_embedding):
    idx = inputs.astype(jnp.int32).reshape(B * L)
    pairs = embedding.reshape(-1, 2 * H)
    prefill = jnp.concatenate(
        [position_embedding[:L], jnp.zeros((L, H), jnp.float32)], axis=1
    )
    return _embed(idx, pairs, prefill)
